# baseline (speedup 1.0000x reference)
import functools

import numpy as np
import jax
import jax.numpy as jnp
from jax import lax
from jax.experimental import pallas as pl
from jax.experimental.pallas import tpu as pltpu
from jax.experimental.pallas import tpu_sc as plsc

_F = 26
_D = 16
_FIELD_SIZE = 38461
_OFFSETS = np.concatenate(
    [[0], np.cumsum([_FIELD_SIZE] * _F)[:-1]]).astype(np.int32)

_NC = 2
_NS = 16
_NW = _NC * _NS
_C = 64
_IPB = _C * _F
_NCHUNK = _IPB // 128


_NROWS = 999987
_FULL = 7812
_TAIL0 = _FULL * 128
_TAILN = _NROWS - _TAIL0
_ROWS_OUT = 1000000
_MCOLS = 1024
_MPW = 30


@functools.cache
def _build_repack():
    mesh = plsc.VectorSubcoreMesh(core_axis_name="c", subcore_axis_name="s",
                                  num_cores=_NC, num_subcores=_NS)

    def body(embT_hbm, out_hbm, inv0, inv1, outv0, outv1, pinv, poutv, tinv,
             sin0, sin1, sout0, sout1):
        cid = lax.axis_index("c")
        sid = lax.axis_index("s")
        wid = sid * _NC + cid
        iota = lax.iota(jnp.int32, 16)

        def transpose_macro(inv, outv, ncols):
            @plsc.parallel_loop(0, ncols, unroll=32)
            def _(j):
                jv = jnp.zeros((16,), jnp.int32) + j
                outv[pl.ds(j * 16, 16)] = plsc.load_gather(inv, [iota, jv])

        def in_cp(buf, sem, m):
            return pltpu.make_async_copy(
                embT_hbm.at[:, pl.ds(m * _MCOLS, _MCOLS)], buf, sem)

        def out_cp(buf, sem, m):
            return pltpu.make_async_copy(
                buf, out_hbm.at[pl.ds(m * (_MCOLS * 16), _MCOLS * 16)], sem)

        def macro_m(j):
            return wid + _NW * j

        in_cp(inv0, sin0, macro_m(0)).start()

        def pipe(i, carry):
            m0 = macro_m(2 * i)
            m1 = macro_m(2 * i + 1)
            in_cp(inv0, sin0, m0).wait()
            in_cp(inv1, sin1, m1).start()

            @pl.when(i > 0)
            def _():
                out_cp(outv0, sout0, m0).wait()
            transpose_macro(inv0, outv0, _MCOLS)
            out_cp(outv0, sout0, m0).start()

            in_cp(inv1, sin1, m1).wait()

            @pl.when(i < (_MPW // 2 - 1))
            def _():
                in_cp(inv0, sin0, macro_m(2 * i + 2)).start()

            @pl.when(i > 0)
            def _():
                out_cp(outv1, sout1, m1).wait()
            transpose_macro(inv1, outv1, _MCOLS)
            out_cp(outv1, sout1, m1).start()
            return carry

        lax.fori_loop(0, _MPW // 2, pipe, jnp.int32(0))
        out_cp(outv0, sout0, macro_m(_MPW - 2)).wait()
        out_cp(outv1, sout1, macro_m(_MPW - 1)).wait()

        @pl.when(wid < 16)
        def _():
            m = wid + 960
            pltpu.sync_copy(embT_hbm.at[:, pl.ds(m * _MCOLS, _MCOLS)], inv0)
            transpose_macro(inv0, outv0, _MCOLS)
            pltpu.sync_copy(outv0,
                            out_hbm.at[pl.ds(m * (_MCOLS * 16), _MCOLS * 16)])

        @pl.when(wid == 16)
        def _():
            c0 = 976 * _MCOLS
            pltpu.sync_copy(embT_hbm.at[:, pl.ds(c0, 512)], pinv)

            @plsc.parallel_loop(0, 512, unroll=16)
            def _(j):
                jv = jnp.zeros((16,), jnp.int32) + j
                poutv[pl.ds(j * 16, 16)] = plsc.load_gather(pinv, [iota, jv])

            pltpu.sync_copy(poutv, out_hbm.at[pl.ds(c0 * 16, 512 * 16)])

        @pl.when(wid == 0)
        def _():
            pltpu.sync_copy(embT_hbm.at[:, pl.ds(_TAIL0, _TAILN)], tinv)
            for j in range(_TAILN):
                jv = jnp.zeros((16,), jnp.int32) + j
                poutv[pl.ds(j * 16, 16)] = plsc.load_gather(tinv, [iota, jv])
            pltpu.sync_copy(poutv.at[pl.ds(0, _TAILN * 16)],
                            out_hbm.at[pl.ds(_TAIL0 * 16, _TAILN * 16)])

    return pl.kernel(
        body,
        out_type=jax.ShapeDtypeStruct((_ROWS_OUT * 16,), jnp.float32),
        mesh=mesh,
        scratch_types=[
            pltpu.VMEM((16, _MCOLS), jnp.float32),
            pltpu.VMEM((16, _MCOLS), jnp.float32),
            pltpu.VMEM((_MCOLS * 16,), jnp.float32),
            pltpu.VMEM((_MCOLS * 16,), jnp.float32),
            pltpu.VMEM((16, 512), jnp.float32),
            pltpu.VMEM((512 * 16,), jnp.float32),
            pltpu.VMEM((16, _TAILN), jnp.float32),
            pltpu.SemaphoreType.DMA,
            pltpu.SemaphoreType.DMA,
            pltpu.SemaphoreType.DMA,
            pltpu.SemaphoreType.DMA,
        ],
        compiler_params=pltpu.CompilerParams(
            needs_layout_passes=False, use_tc_tiling_on_sc=True),
    )


@functools.cache
def _build(batch):
    assert batch % (_NW * _C) == 0
    b_per_w = batch // _NW
    nblk = b_per_w // _C
    mesh = plsc.VectorSubcoreMesh(core_axis_name="c", subcore_axis_name="s",
                                  num_cores=_NC, num_subcores=_NS)

    def body(x_hbm, lin_hbm, emb_hbm, bias_hbm, off_hbm, out_hbm,
             xv, idxv, offv, biasv, rowsv, linv, outv, sem):
        cid = lax.axis_index("c")
        sid = lax.axis_index("s")
        wid = sid * _NC + cid
        base_row = wid * b_per_w

        pltpu.sync_copy(off_hbm, offv)
        pltpu.sync_copy(bias_hbm, biasv)
        biasvec = biasv[...]
        iota = lax.iota(jnp.int32, 16)

        def blk_body(blk, carry):
            row0 = base_row + blk * _C
            pltpu.sync_copy(x_hbm.at[pl.ds(row0 * _F, _IPB)], xv)

            @plsc.parallel_loop(0, _IPB // 16, unroll=8)
            def _(t):
                sl = pl.ds(t * 16, 16)
                idxv[sl] = xv[sl] + offv[sl]

            copies = []
            for j in range(_NCHUNK):
                copies.append(pltpu.make_async_copy(
                    emb_hbm.at[idxv.at[pl.ds(j * 128, 128)]],
                    rowsv.at[pl.ds(j * 128, 128)], sem))
                copies.append(pltpu.make_async_copy(
                    lin_hbm.at[idxv.at[pl.ds(j * 128, 128)]],
                    linv.at[pl.ds(j * 128, 128)], sem))
            for c in copies:
                c.start()
            for c in copies:
                c.wait()

            for g in range(_C // 16):
                rbase = iota * _F + g * (16 * _F)

                @plsc.parallel_loop(0, _F, unroll=8,
                                    carry=jnp.zeros((16,), jnp.float32))
                def linsum(f, part):
                    return part + plsc.load_gather(linv, [rbase + f])

                @plsc.parallel_loop(0, _D, unroll=4,
                                    carry=jnp.zeros((16,), jnp.float32))
                def acc(d, part):
                    dvec = jnp.zeros((16,), jnp.int32) + d
                    s = jnp.zeros((16,), jnp.float32)
                    sq = jnp.zeros((16,), jnp.float32)
                    for f in range(_F):
                        e = plsc.load_gather(rowsv, [rbase + f, dvec])
                        s = s + e
                        sq = sq + e * e
                    return part + (s * s - sq)

                z = biasvec + linsum + 0.5 * acc
                outv[pl.ds(g * 16, 16)] = 1.0 / (1.0 + jnp.exp(-z))

            pltpu.sync_copy(outv, out_hbm.at[pl.ds(row0, _C)])
            return carry

        lax.fori_loop(0, nblk, blk_body, jnp.int32(0))

    return pl.kernel(
        body,
        out_type=jax.ShapeDtypeStruct((batch,), jnp.float32),
        mesh=mesh,
        scratch_types=[
            pltpu.VMEM((_IPB,), jnp.int32),
            pltpu.VMEM((_IPB,), jnp.int32),
            pltpu.VMEM((_IPB,), jnp.int32),
            pltpu.VMEM((16,), jnp.float32),
            pltpu.VMEM((_IPB, _D), jnp.float32),
            pltpu.VMEM((_IPB,), jnp.float32),
            pltpu.VMEM((_C,), jnp.float32),
            pltpu.SemaphoreType.DMA,
        ],
        compiler_params=pltpu.CompilerParams(
            needs_layout_passes=False, use_tc_tiling_on_sc=False),
    )


def kernel(x, linear_w, embed_w, bias):
    batch, nf = x.shape
    assert nf == _F
    x_flat = x.reshape(-1).astype(jnp.int32)
    lin_flat = linear_w.reshape(-1).astype(jnp.float32)
    bias16 = jnp.broadcast_to(bias.reshape(()), (16,)).astype(jnp.float32)
    off_tile = jnp.asarray(np.tile(_OFFSETS, _C))
    table = _build_repack()(embed_w.T).reshape(_ROWS_OUT, 16)
    out = _build(batch)(x_flat, lin_flat, table, bias16, off_tile)
    return out.reshape(batch, 1)


# speedup vs baseline: 1.0977x; 1.0977x over previous
import functools

import numpy as np
import jax
import jax.numpy as jnp
from jax import lax
from jax.experimental import pallas as pl
from jax.experimental.pallas import tpu as pltpu
from jax.experimental.pallas import tpu_sc as plsc

_F = 26
_D = 16
_FIELD_SIZE = 38461
_OFFSETS = np.concatenate(
    [[0], np.cumsum([_FIELD_SIZE] * _F)[:-1]]).astype(np.int32)

_NC = 2
_NS = 16
_NW = _NC * _NS
_C = 64
_IPB = _C * _F
_NCHUNK = _IPB // 128


_NROWS = 999987
_FULL = 7812
_TAIL0 = _FULL * 128
_TAILN = _NROWS - _TAIL0
_ROWS_OUT = 1000000
_MCOLS = 1024
_MPW = 30


@functools.cache
def _build_repack():
    mesh = plsc.VectorSubcoreMesh(core_axis_name="c", subcore_axis_name="s",
                                  num_cores=_NC, num_subcores=_NS)

    def body(embT_hbm, out_hbm, inv0, inv1, outv0, outv1, pinv, poutv, tinv,
             sin0, sin1, sout0, sout1):
        cid = lax.axis_index("c")
        sid = lax.axis_index("s")
        wid = sid * _NC + cid
        iota = lax.iota(jnp.int32, 16)

        def transpose_macro(inv, outv, ncols):
            @plsc.parallel_loop(0, ncols, unroll=16)
            def _(j):
                jv = jnp.zeros((16,), jnp.int32) + j
                outv[pl.ds(j * 16, 16)] = plsc.load_gather(inv, [iota, jv])

        def in_cp(buf, sem, m):
            return pltpu.make_async_copy(
                embT_hbm.at[:, pl.ds(m * _MCOLS, _MCOLS)], buf, sem)

        def out_cp(buf, sem, m):
            return pltpu.make_async_copy(
                buf, out_hbm.at[pl.ds(m * (_MCOLS * 16), _MCOLS * 16)], sem)

        def macro_m(j):
            return wid + _NW * j

        in_cp(inv0, sin0, macro_m(0)).start()

        def pipe(i, carry):
            m0 = macro_m(2 * i)
            m1 = macro_m(2 * i + 1)
            in_cp(inv0, sin0, m0).wait()
            in_cp(inv1, sin1, m1).start()

            @pl.when(i > 0)
            def _():
                out_cp(outv0, sout0, m0).wait()
            transpose_macro(inv0, outv0, _MCOLS)
            out_cp(outv0, sout0, m0).start()

            in_cp(inv1, sin1, m1).wait()

            @pl.when(i < (_MPW // 2 - 1))
            def _():
                in_cp(inv0, sin0, macro_m(2 * i + 2)).start()

            @pl.when(i > 0)
            def _():
                out_cp(outv1, sout1, m1).wait()
            transpose_macro(inv1, outv1, _MCOLS)
            out_cp(outv1, sout1, m1).start()
            return carry

        lax.fori_loop(0, _MPW // 2, pipe, jnp.int32(0))
        out_cp(outv0, sout0, macro_m(_MPW - 2)).wait()
        out_cp(outv1, sout1, macro_m(_MPW - 1)).wait()

        @pl.when(wid < 16)
        def _():
            m = wid + 960
            pltpu.sync_copy(embT_hbm.at[:, pl.ds(m * _MCOLS, _MCOLS)], inv0)
            transpose_macro(inv0, outv0, _MCOLS)
            pltpu.sync_copy(outv0,
                            out_hbm.at[pl.ds(m * (_MCOLS * 16), _MCOLS * 16)])

        @pl.when(wid == 16)
        def _():
            c0 = 976 * _MCOLS
            pltpu.sync_copy(embT_hbm.at[:, pl.ds(c0, 512)], pinv)

            @plsc.parallel_loop(0, 512, unroll=16)
            def _(j):
                jv = jnp.zeros((16,), jnp.int32) + j
                poutv[pl.ds(j * 16, 16)] = plsc.load_gather(pinv, [iota, jv])

            pltpu.sync_copy(poutv, out_hbm.at[pl.ds(c0 * 16, 512 * 16)])

        @pl.when(wid == 0)
        def _():
            pltpu.sync_copy(embT_hbm.at[:, pl.ds(_TAIL0, _TAILN)], tinv)
            for j in range(_TAILN):
                jv = jnp.zeros((16,), jnp.int32) + j
                poutv[pl.ds(j * 16, 16)] = plsc.load_gather(tinv, [iota, jv])
            pltpu.sync_copy(poutv.at[pl.ds(0, _TAILN * 16)],
                            out_hbm.at[pl.ds(_TAIL0 * 16, _TAILN * 16)])

    return pl.kernel(
        body,
        out_type=jax.ShapeDtypeStruct((_ROWS_OUT * 16,), jnp.float32),
        mesh=mesh,
        scratch_types=[
            pltpu.VMEM((16, _MCOLS), jnp.float32),
            pltpu.VMEM((16, _MCOLS), jnp.float32),
            pltpu.VMEM((_MCOLS * 16,), jnp.float32),
            pltpu.VMEM((_MCOLS * 16,), jnp.float32),
            pltpu.VMEM((16, 512), jnp.float32),
            pltpu.VMEM((512 * 16,), jnp.float32),
            pltpu.VMEM((16, _TAILN), jnp.float32),
            pltpu.SemaphoreType.DMA,
            pltpu.SemaphoreType.DMA,
            pltpu.SemaphoreType.DMA,
            pltpu.SemaphoreType.DMA,
        ],
        compiler_params=pltpu.CompilerParams(
            needs_layout_passes=False, use_tc_tiling_on_sc=True),
    )


@functools.cache
def _build(batch):
    assert batch % (_NW * _C) == 0
    b_per_w = batch // _NW
    nblk = b_per_w // _C
    mesh = plsc.VectorSubcoreMesh(core_axis_name="c", subcore_axis_name="s",
                                  num_cores=_NC, num_subcores=_NS)

    def body(x_hbm, lin_hbm, emb_hbm, bias_hbm, off_hbm, out_hbm,
             xv, idxv, offv, biasv, rowsv, linv, outv, sem):
        cid = lax.axis_index("c")
        sid = lax.axis_index("s")
        wid = sid * _NC + cid
        base_row = wid * b_per_w

        pltpu.sync_copy(off_hbm, offv)
        pltpu.sync_copy(bias_hbm, biasv)
        biasvec = biasv[...]
        iota = lax.iota(jnp.int32, 16)

        def blk_body(blk, carry):
            row0 = base_row + blk * _C
            pltpu.sync_copy(x_hbm.at[pl.ds(row0 * _F, _IPB)], xv)

            @plsc.parallel_loop(0, _IPB // 16, unroll=8)
            def _(t):
                sl = pl.ds(t * 16, 16)
                idxv[sl] = xv[sl] + offv[sl]

            copies = []
            for j in range(_NCHUNK):
                copies.append(pltpu.make_async_copy(
                    emb_hbm.at[idxv.at[pl.ds(j * 128, 128)]],
                    rowsv.at[pl.ds(j * 128, 128)], sem))
                copies.append(pltpu.make_async_copy(
                    lin_hbm.at[idxv.at[pl.ds(j * 128, 128)]],
                    linv.at[pl.ds(j * 128, 128)], sem))
            for c in copies:
                c.start()
            for c in copies:
                c.wait()

            for g in range(_C // 16):
                rbase = iota * _F + g * (16 * _F)

                linsum = jnp.zeros((16,), jnp.float32)
                for f in range(_F):
                    linsum = linsum + plsc.load_gather(linv, [rbase + f])

                def d_body(d, part):
                    dvec = jnp.zeros((16,), jnp.int32) + d
                    s = jnp.zeros((16,), jnp.float32)
                    sq = jnp.zeros((16,), jnp.float32)
                    for f in range(_F):
                        e = plsc.load_gather(rowsv, [rbase + f, dvec])
                        s = s + e
                        sq = sq + e * e
                    return part + (s * s - sq)

                acc = lax.fori_loop(0, _D, d_body,
                                    jnp.zeros((16,), jnp.float32))

                z = biasvec + linsum + 0.5 * acc
                outv[pl.ds(g * 16, 16)] = 1.0 / (1.0 + jnp.exp(-z))

            pltpu.sync_copy(outv, out_hbm.at[pl.ds(row0, _C)])
            return carry

        lax.fori_loop(0, nblk, blk_body, jnp.int32(0))

    return pl.kernel(
        body,
        out_type=jax.ShapeDtypeStruct((batch,), jnp.float32),
        mesh=mesh,
        scratch_types=[
            pltpu.VMEM((_IPB,), jnp.int32),
            pltpu.VMEM((_IPB,), jnp.int32),
            pltpu.VMEM((_IPB,), jnp.int32),
            pltpu.VMEM((16,), jnp.float32),
            pltpu.VMEM((_IPB, _D), jnp.float32),
            pltpu.VMEM((_IPB,), jnp.float32),
            pltpu.VMEM((_C,), jnp.float32),
            pltpu.SemaphoreType.DMA,
        ],
        compiler_params=pltpu.CompilerParams(
            needs_layout_passes=False, use_tc_tiling_on_sc=False),
    )


def kernel(x, linear_w, embed_w, bias):
    batch, nf = x.shape
    assert nf == _F
    x_flat = x.reshape(-1).astype(jnp.int32)
    lin_flat = linear_w.reshape(-1).astype(jnp.float32)
    bias16 = jnp.broadcast_to(bias.reshape(()), (16,)).astype(jnp.float32)
    off_tile = jnp.asarray(np.tile(_OFFSETS, _C))
    table = _build_repack()(embed_w.T).reshape(_ROWS_OUT, 16)
    out = _build(batch)(x_flat, lin_flat, table, bias16, off_tile)
    return out.reshape(batch, 1)


# speedup vs baseline: 1.3531x; 1.2327x over previous
import functools

import numpy as np
import jax
import jax.numpy as jnp
from jax import lax
from jax.experimental import pallas as pl
from jax.experimental.pallas import tpu as pltpu
from jax.experimental.pallas import tpu_sc as plsc

_F = 26
_D = 16
_FIELD_SIZE = 38461
_OFFSETS = np.concatenate(
    [[0], np.cumsum([_FIELD_SIZE] * _F)[:-1]]).astype(np.int32)

_NC = 2
_NS = 16
_NW = _NC * _NS
_C = 64
_IPB = _C * _F
_NCHUNK = _IPB // 128


_NROWS = 999987
_FULL = 7812
_TAIL0 = _FULL * 128
_TAILN = _NROWS - _TAIL0
_ROWS_OUT = 1000000
_MCOLS = 1024
_MPW = 30


@functools.cache
def _build_repack():
    mesh = plsc.VectorSubcoreMesh(core_axis_name="c", subcore_axis_name="s",
                                  num_cores=_NC, num_subcores=_NS)

    def body(embT_hbm, out_hbm, inv0, inv1, outv0, outv1, pinv, poutv, tinv,
             sin0, sin1, sout0, sout1):
        cid = lax.axis_index("c")
        sid = lax.axis_index("s")
        wid = sid * _NC + cid
        iota = lax.iota(jnp.int32, 16)

        def transpose_macro(inv, outv, ncols):
            @plsc.parallel_loop(0, ncols, unroll=16)
            def _(j):
                jv = jnp.zeros((16,), jnp.int32) + j
                outv[pl.ds(j * 16, 16)] = plsc.load_gather(inv, [iota, jv])

        def in_cp(buf, sem, m):
            return pltpu.make_async_copy(
                embT_hbm.at[:, pl.ds(m * _MCOLS, _MCOLS)],
                buf.at[:, pl.ds(0, _MCOLS)], sem)

        def out_cp(buf, sem, m):
            return pltpu.make_async_copy(
                buf, out_hbm.at[pl.ds(m * (_MCOLS * 16), _MCOLS * 16)], sem)

        def macro_m(j):
            return wid + _NW * j

        in_cp(inv0, sin0, macro_m(0)).start()

        def pipe(i, carry):
            m0 = macro_m(2 * i)
            m1 = macro_m(2 * i + 1)
            in_cp(inv0, sin0, m0).wait()
            in_cp(inv1, sin1, m1).start()

            @pl.when(i > 0)
            def _():
                out_cp(outv0, sout0, m0).wait()
            transpose_macro(inv0, outv0, _MCOLS)
            out_cp(outv0, sout0, m0).start()

            in_cp(inv1, sin1, m1).wait()

            @pl.when(i < (_MPW // 2 - 1))
            def _():
                in_cp(inv0, sin0, macro_m(2 * i + 2)).start()

            @pl.when(i > 0)
            def _():
                out_cp(outv1, sout1, m1).wait()
            transpose_macro(inv1, outv1, _MCOLS)
            out_cp(outv1, sout1, m1).start()
            return carry

        lax.fori_loop(0, _MPW // 2, pipe, jnp.int32(0))
        out_cp(outv0, sout0, macro_m(_MPW - 2)).wait()
        out_cp(outv1, sout1, macro_m(_MPW - 1)).wait()

        @pl.when(wid < 16)
        def _():
            m = wid + 960
            pltpu.sync_copy(embT_hbm.at[:, pl.ds(m * _MCOLS, _MCOLS)],
                            inv0.at[:, pl.ds(0, _MCOLS)])
            transpose_macro(inv0, outv0, _MCOLS)
            pltpu.sync_copy(outv0,
                            out_hbm.at[pl.ds(m * (_MCOLS * 16), _MCOLS * 16)])

        @pl.when(wid == 16)
        def _():
            c0 = 976 * _MCOLS
            pltpu.sync_copy(embT_hbm.at[:, pl.ds(c0, 512)],
                            pinv.at[:, pl.ds(0, 512)])

            @plsc.parallel_loop(0, 512, unroll=16)
            def _(j):
                jv = jnp.zeros((16,), jnp.int32) + j
                poutv[pl.ds(j * 16, 16)] = plsc.load_gather(pinv, [iota, jv])

            pltpu.sync_copy(poutv, out_hbm.at[pl.ds(c0 * 16, 512 * 16)])

        @pl.when(wid == 0)
        def _():
            pltpu.sync_copy(embT_hbm.at[:, pl.ds(_TAIL0, _TAILN)], tinv)
            for j in range(_TAILN):
                jv = jnp.zeros((16,), jnp.int32) + j
                poutv[pl.ds(j * 16, 16)] = plsc.load_gather(tinv, [iota, jv])
            pltpu.sync_copy(poutv.at[pl.ds(0, _TAILN * 16)],
                            out_hbm.at[pl.ds(_TAIL0 * 16, _TAILN * 16)])

    return pl.kernel(
        body,
        out_type=jax.ShapeDtypeStruct((_ROWS_OUT * 16,), jnp.float32),
        mesh=mesh,
        scratch_types=[
            pltpu.VMEM((16, _MCOLS + 1), jnp.float32),
            pltpu.VMEM((16, _MCOLS + 1), jnp.float32),
            pltpu.VMEM((_MCOLS * 16,), jnp.float32),
            pltpu.VMEM((_MCOLS * 16,), jnp.float32),
            pltpu.VMEM((16, 513), jnp.float32),
            pltpu.VMEM((512 * 16,), jnp.float32),
            pltpu.VMEM((16, _TAILN), jnp.float32),
            pltpu.SemaphoreType.DMA,
            pltpu.SemaphoreType.DMA,
            pltpu.SemaphoreType.DMA,
            pltpu.SemaphoreType.DMA,
        ],
        compiler_params=pltpu.CompilerParams(
            needs_layout_passes=False, use_tc_tiling_on_sc=True),
    )


@functools.cache
def _build(batch):
    assert batch % (_NW * _C) == 0
    b_per_w = batch // _NW
    nblk = b_per_w // _C
    mesh = plsc.VectorSubcoreMesh(core_axis_name="c", subcore_axis_name="s",
                                  num_cores=_NC, num_subcores=_NS)

    def body(x_hbm, lin_hbm, emb_hbm, bias_hbm, off_hbm, out_hbm,
             xv, idxv, offv, biasv, rowsv, linv, outv, sem):
        cid = lax.axis_index("c")
        sid = lax.axis_index("s")
        wid = sid * _NC + cid
        base_row = wid * b_per_w

        pltpu.sync_copy(off_hbm, offv)
        pltpu.sync_copy(bias_hbm, biasv)
        biasvec = biasv[...]
        iota = lax.iota(jnp.int32, 16)

        def blk_body(blk, carry):
            row0 = base_row + blk * _C
            pltpu.sync_copy(x_hbm.at[pl.ds(row0 * _F, _IPB)], xv)

            @plsc.parallel_loop(0, _IPB // 16, unroll=8)
            def _(t):
                sl = pl.ds(t * 16, 16)
                idxv[sl] = xv[sl] + offv[sl]

            copies = []
            for j in range(_NCHUNK):
                copies.append(pltpu.make_async_copy(
                    emb_hbm.at[idxv.at[pl.ds(j * 128, 128)]],
                    rowsv.at[pl.ds(j * 128, 128)], sem))
                copies.append(pltpu.make_async_copy(
                    lin_hbm.at[idxv.at[pl.ds(j * 128, 128)]],
                    linv.at[pl.ds(j * 128, 128)], sem))
            for c in copies:
                c.start()
            for c in copies:
                c.wait()

            for g in range(_C // 16):
                rbase = iota * _F + g * (16 * _F)

                linsum = jnp.zeros((16,), jnp.float32)
                for f in range(_F):
                    linsum = linsum + plsc.load_gather(linv, [rbase + f])

                def d_body(d, part):
                    dvec = (iota + d) & 15
                    s = jnp.zeros((16,), jnp.float32)
                    sq = jnp.zeros((16,), jnp.float32)
                    for f in range(_F):
                        e = plsc.load_gather(rowsv, [rbase + f, dvec])
                        s = s + e
                        sq = sq + e * e
                    return part + (s * s - sq)

                acc = lax.fori_loop(0, _D, d_body,
                                    jnp.zeros((16,), jnp.float32))

                z = biasvec + linsum + 0.5 * acc
                outv[pl.ds(g * 16, 16)] = 1.0 / (1.0 + jnp.exp(-z))

            pltpu.sync_copy(outv, out_hbm.at[pl.ds(row0, _C)])
            return carry

        lax.fori_loop(0, nblk, blk_body, jnp.int32(0))

    return pl.kernel(
        body,
        out_type=jax.ShapeDtypeStruct((batch,), jnp.float32),
        mesh=mesh,
        scratch_types=[
            pltpu.VMEM((_IPB,), jnp.int32),
            pltpu.VMEM((_IPB,), jnp.int32),
            pltpu.VMEM((_IPB,), jnp.int32),
            pltpu.VMEM((16,), jnp.float32),
            pltpu.VMEM((_IPB, _D), jnp.float32),
            pltpu.VMEM((_IPB,), jnp.float32),
            pltpu.VMEM((_C,), jnp.float32),
            pltpu.SemaphoreType.DMA,
        ],
        compiler_params=pltpu.CompilerParams(
            needs_layout_passes=False, use_tc_tiling_on_sc=False),
    )


def kernel(x, linear_w, embed_w, bias):
    batch, nf = x.shape
    assert nf == _F
    x_flat = x.reshape(-1).astype(jnp.int32)
    lin_flat = linear_w.reshape(-1).astype(jnp.float32)
    bias16 = jnp.broadcast_to(bias.reshape(()), (16,)).astype(jnp.float32)
    off_tile = jnp.asarray(np.tile(_OFFSETS, _C))
    table = _build_repack()(embed_w.T).reshape(_ROWS_OUT, 16)
    out = _build(batch)(x_flat, lin_flat, table, bias16, off_tile)
    return out.reshape(batch, 1)


# speedup vs baseline: 2.7776x; 2.0528x over previous
import functools

import numpy as np
import jax
import jax.numpy as jnp
from jax import lax
from jax.experimental import pallas as pl
from jax.experimental.pallas import tpu as pltpu
from jax.experimental.pallas import tpu_sc as plsc

_F = 26
_D = 16
_FIELD_SIZE = 38461
_OFFSETS = np.concatenate(
    [[0], np.cumsum([_FIELD_SIZE] * _F)[:-1]]).astype(np.int32)

_NC = 2
_NS = 16
_NW = _NC * _NS
_C = 64
_IPB = _C * _F
_NCHUNK = _IPB // 128


_NROWS = 999987
_FULL = 7812
_TAIL0 = _FULL * 128
_TAILN = _NROWS - _TAIL0
_ROWS_OUT = 1000000
_MCOLS = 1024
_MPW = 30


@functools.cache
def _build_repack():
    mesh = plsc.VectorSubcoreMesh(core_axis_name="c", subcore_axis_name="s",
                                  num_cores=_NC, num_subcores=_NS)

    def body(embT_hbm, out_hbm, inv0, inv1, outv0, outv1, pinv, poutv, tinv,
             sin0, sin1, sout0, sout1):
        cid = lax.axis_index("c")
        sid = lax.axis_index("s")
        wid = sid * _NC + cid
        iota = lax.iota(jnp.int32, 16)

        def transpose_macro(inv, outv, ncols):
            @plsc.parallel_loop(0, ncols, unroll=16)
            def _(j):
                cvec = (jnp.zeros((16,), jnp.int32) + j + iota) & (ncols - 1)
                e = plsc.load_gather(inv, [iota, cvec])
                plsc.store_scatter(outv, [cvec * 16 + iota], e)

        def in_cp(buf, sem, m):
            return pltpu.make_async_copy(
                embT_hbm.at[:, pl.ds(m * _MCOLS, _MCOLS)], buf, sem)

        def out_cp(buf, sem, m):
            return pltpu.make_async_copy(
                buf, out_hbm.at[pl.ds(m * (_MCOLS * 16), _MCOLS * 16)], sem)

        def macro_m(j):
            return wid + _NW * j

        in_cp(inv0, sin0, macro_m(0)).start()

        def pipe(i, carry):
            m0 = macro_m(2 * i)
            m1 = macro_m(2 * i + 1)
            in_cp(inv0, sin0, m0).wait()
            in_cp(inv1, sin1, m1).start()

            @pl.when(i > 0)
            def _():
                out_cp(outv0, sout0, m0).wait()
            transpose_macro(inv0, outv0, _MCOLS)
            out_cp(outv0, sout0, m0).start()

            in_cp(inv1, sin1, m1).wait()

            @pl.when(i < (_MPW // 2 - 1))
            def _():
                in_cp(inv0, sin0, macro_m(2 * i + 2)).start()

            @pl.when(i > 0)
            def _():
                out_cp(outv1, sout1, m1).wait()
            transpose_macro(inv1, outv1, _MCOLS)
            out_cp(outv1, sout1, m1).start()
            return carry

        lax.fori_loop(0, _MPW // 2, pipe, jnp.int32(0))
        out_cp(outv0, sout0, macro_m(_MPW - 2)).wait()
        out_cp(outv1, sout1, macro_m(_MPW - 1)).wait()

        @pl.when(wid < 16)
        def _():
            m = wid + 960
            pltpu.sync_copy(embT_hbm.at[:, pl.ds(m * _MCOLS, _MCOLS)], inv0)
            transpose_macro(inv0, outv0, _MCOLS)
            pltpu.sync_copy(outv0,
                            out_hbm.at[pl.ds(m * (_MCOLS * 16), _MCOLS * 16)])

        @pl.when(wid == 16)
        def _():
            c0 = 976 * _MCOLS
            pltpu.sync_copy(embT_hbm.at[:, pl.ds(c0, 512)], pinv)
            transpose_macro(pinv, poutv, 512)
            pltpu.sync_copy(poutv, out_hbm.at[pl.ds(c0 * 16, 512 * 16)])

        @pl.when(wid == 0)
        def _():
            pltpu.sync_copy(embT_hbm.at[:, pl.ds(_TAIL0, _TAILN)], tinv)
            for j in range(_TAILN):
                jv = jnp.zeros((16,), jnp.int32) + j
                poutv[pl.ds(j * 16, 16)] = plsc.load_gather(tinv, [iota, jv])
            pltpu.sync_copy(poutv.at[pl.ds(0, _TAILN * 16)],
                            out_hbm.at[pl.ds(_TAIL0 * 16, _TAILN * 16)])

    return pl.kernel(
        body,
        out_type=jax.ShapeDtypeStruct((_ROWS_OUT * 16,), jnp.float32),
        mesh=mesh,
        scratch_types=[
            pltpu.VMEM((16, _MCOLS), jnp.float32),
            pltpu.VMEM((16, _MCOLS), jnp.float32),
            pltpu.VMEM((_MCOLS * 16,), jnp.float32),
            pltpu.VMEM((_MCOLS * 16,), jnp.float32),
            pltpu.VMEM((16, 512), jnp.float32),
            pltpu.VMEM((512 * 16,), jnp.float32),
            pltpu.VMEM((16, _TAILN), jnp.float32),
            pltpu.SemaphoreType.DMA,
            pltpu.SemaphoreType.DMA,
            pltpu.SemaphoreType.DMA,
            pltpu.SemaphoreType.DMA,
        ],
        compiler_params=pltpu.CompilerParams(
            needs_layout_passes=False, use_tc_tiling_on_sc=True),
    )


@functools.cache
def _build(batch):
    assert batch % (_NW * _C) == 0
    b_per_w = batch // _NW
    nblk = b_per_w // _C
    mesh = plsc.VectorSubcoreMesh(core_axis_name="c", subcore_axis_name="s",
                                  num_cores=_NC, num_subcores=_NS)

    def body(x_hbm, lin_hbm, emb_hbm, bias_hbm, off_hbm, out_hbm,
             xv, idxv, offv, biasv, rowsv, linv, outv, sem):
        cid = lax.axis_index("c")
        sid = lax.axis_index("s")
        wid = sid * _NC + cid
        base_row = wid * b_per_w

        pltpu.sync_copy(off_hbm, offv)
        pltpu.sync_copy(bias_hbm, biasv)
        biasvec = biasv[...]
        iota = lax.iota(jnp.int32, 16)

        def blk_body(blk, carry):
            row0 = base_row + blk * _C
            pltpu.sync_copy(x_hbm.at[pl.ds(row0 * _F, _IPB)], xv)

            @plsc.parallel_loop(0, _IPB // 16, unroll=8)
            def _(t):
                sl = pl.ds(t * 16, 16)
                idxv[sl] = xv[sl] + offv[sl]

            copies = []
            for j in range(_NCHUNK):
                copies.append(pltpu.make_async_copy(
                    emb_hbm.at[idxv.at[pl.ds(j * 128, 128)]],
                    rowsv.at[pl.ds(j * 128, 128)], sem))
                copies.append(pltpu.make_async_copy(
                    lin_hbm.at[idxv.at[pl.ds(j * 128, 128)]],
                    linv.at[pl.ds(j * 128, 128)], sem))
            for c in copies:
                c.start()
            for c in copies:
                c.wait()

            for g in range(_C // 16):
                rbase = iota * _F + g * (16 * _F)

                linsum = jnp.zeros((16,), jnp.float32)
                for f in range(_F):
                    linsum = linsum + plsc.load_gather(linv, [rbase + f])

                def d_body(d, part):
                    dvec = (iota + d) & 15
                    s = jnp.zeros((16,), jnp.float32)
                    sq = jnp.zeros((16,), jnp.float32)
                    for f in range(_F):
                        e = plsc.load_gather(rowsv, [rbase + f, dvec])
                        s = s + e
                        sq = sq + e * e
                    return part + (s * s - sq)

                acc = lax.fori_loop(0, _D, d_body,
                                    jnp.zeros((16,), jnp.float32))

                z = biasvec + linsum + 0.5 * acc
                outv[pl.ds(g * 16, 16)] = 1.0 / (1.0 + jnp.exp(-z))

            pltpu.sync_copy(outv, out_hbm.at[pl.ds(row0, _C)])
            return carry

        lax.fori_loop(0, nblk, blk_body, jnp.int32(0))

    return pl.kernel(
        body,
        out_type=jax.ShapeDtypeStruct((batch,), jnp.float32),
        mesh=mesh,
        scratch_types=[
            pltpu.VMEM((_IPB,), jnp.int32),
            pltpu.VMEM((_IPB,), jnp.int32),
            pltpu.VMEM((_IPB,), jnp.int32),
            pltpu.VMEM((16,), jnp.float32),
            pltpu.VMEM((_IPB, _D), jnp.float32),
            pltpu.VMEM((_IPB,), jnp.float32),
            pltpu.VMEM((_C,), jnp.float32),
            pltpu.SemaphoreType.DMA,
        ],
        compiler_params=pltpu.CompilerParams(
            needs_layout_passes=False, use_tc_tiling_on_sc=False),
    )


def kernel(x, linear_w, embed_w, bias):
    batch, nf = x.shape
    assert nf == _F
    x_flat = x.reshape(-1).astype(jnp.int32)
    lin_flat = linear_w.reshape(-1).astype(jnp.float32)
    bias16 = jnp.broadcast_to(bias.reshape(()), (16,)).astype(jnp.float32)
    off_tile = jnp.asarray(np.tile(_OFFSETS, _C))
    table = _build_repack()(embed_w.T).reshape(_ROWS_OUT, 16)
    out = _build(batch)(x_flat, lin_flat, table, bias16, off_tile)
    return out.reshape(batch, 1)


# speedup vs baseline: 3.1744x; 1.1429x over previous
import functools

import numpy as np
import jax
import jax.numpy as jnp
from jax import lax
from jax.experimental import pallas as pl
from jax.experimental.pallas import tpu as pltpu
from jax.experimental.pallas import tpu_sc as plsc

_F = 26
_D = 16
_FIELD_SIZE = 38461
_OFFSETS = np.concatenate(
    [[0], np.cumsum([_FIELD_SIZE] * _F)[:-1]]).astype(np.int32)

_NC = 2
_NS = 16
_NW = _NC * _NS
_C = 64
_IPB = _C * _F
_NCHUNK = _IPB // 128


_NROWS = 999987
_FULL = 7812
_TAIL0 = _FULL * 128
_TAILN = _NROWS - _TAIL0
_ROWS_OUT = 1000000
_MCOLS = 1024
_MPW = 30


@functools.cache
def _build_repack():
    mesh = plsc.VectorSubcoreMesh(core_axis_name="c", subcore_axis_name="s",
                                  num_cores=_NC, num_subcores=_NS)

    def body(embT_hbm, out_hbm, inv0, inv1, outv0, outv1, pinv, poutv, tinv,
             sin0, sin1, sout0, sout1):
        cid = lax.axis_index("c")
        sid = lax.axis_index("s")
        wid = sid * _NC + cid
        iota = lax.iota(jnp.int32, 16)

        def transpose_macro(inv, outv, ncols):
            @plsc.parallel_loop(0, ncols, unroll=16)
            def _(j):
                cvec = (jnp.zeros((16,), jnp.int32) + j + iota) & (ncols - 1)
                e = plsc.load_gather(inv, [iota, cvec])
                plsc.store_scatter(outv, [cvec * 16 + iota], e)

        def in_cp(buf, sem, m):
            return pltpu.make_async_copy(
                embT_hbm.at[:, pl.ds(m * _MCOLS, _MCOLS)], buf, sem)

        def out_cp(buf, sem, m):
            return pltpu.make_async_copy(
                buf, out_hbm.at[pl.ds(m * (_MCOLS * 16), _MCOLS * 16)], sem)

        def macro_m(j):
            return wid + _NW * j

        in_cp(inv0, sin0, macro_m(0)).start()

        def pipe(i, carry):
            m0 = macro_m(2 * i)
            m1 = macro_m(2 * i + 1)
            in_cp(inv0, sin0, m0).wait()
            in_cp(inv1, sin1, m1).start()

            @pl.when(i > 0)
            def _():
                out_cp(outv0, sout0, m0).wait()
            transpose_macro(inv0, outv0, _MCOLS)
            out_cp(outv0, sout0, m0).start()

            in_cp(inv1, sin1, m1).wait()

            @pl.when(i < (_MPW // 2 - 1))
            def _():
                in_cp(inv0, sin0, macro_m(2 * i + 2)).start()

            @pl.when(i > 0)
            def _():
                out_cp(outv1, sout1, m1).wait()
            transpose_macro(inv1, outv1, _MCOLS)
            out_cp(outv1, sout1, m1).start()
            return carry

        lax.fori_loop(0, _MPW // 2, pipe, jnp.int32(0))
        out_cp(outv0, sout0, macro_m(_MPW - 2)).wait()
        out_cp(outv1, sout1, macro_m(_MPW - 1)).wait()

        @pl.when(wid < 16)
        def _():
            m = wid + 960
            pltpu.sync_copy(embT_hbm.at[:, pl.ds(m * _MCOLS, _MCOLS)], inv0)
            transpose_macro(inv0, outv0, _MCOLS)
            pltpu.sync_copy(outv0,
                            out_hbm.at[pl.ds(m * (_MCOLS * 16), _MCOLS * 16)])

        @pl.when(wid == 16)
        def _():
            c0 = 976 * _MCOLS
            pltpu.sync_copy(embT_hbm.at[:, pl.ds(c0, 512)], pinv)
            transpose_macro(pinv, poutv, 512)
            pltpu.sync_copy(poutv, out_hbm.at[pl.ds(c0 * 16, 512 * 16)])

        @pl.when(wid == 0)
        def _():
            pltpu.sync_copy(embT_hbm.at[:, pl.ds(_TAIL0, _TAILN)], tinv)
            for j in range(_TAILN):
                jv = jnp.zeros((16,), jnp.int32) + j
                poutv[pl.ds(j * 16, 16)] = plsc.load_gather(tinv, [iota, jv])
            pltpu.sync_copy(poutv.at[pl.ds(0, _TAILN * 16)],
                            out_hbm.at[pl.ds(_TAIL0 * 16, _TAILN * 16)])

    return pl.kernel(
        body,
        out_type=jax.ShapeDtypeStruct((_ROWS_OUT * 16,), jnp.float32),
        mesh=mesh,
        scratch_types=[
            pltpu.VMEM((16, _MCOLS), jnp.float32),
            pltpu.VMEM((16, _MCOLS), jnp.float32),
            pltpu.VMEM((_MCOLS * 16,), jnp.float32),
            pltpu.VMEM((_MCOLS * 16,), jnp.float32),
            pltpu.VMEM((16, 512), jnp.float32),
            pltpu.VMEM((512 * 16,), jnp.float32),
            pltpu.VMEM((16, _TAILN), jnp.float32),
            pltpu.SemaphoreType.DMA,
            pltpu.SemaphoreType.DMA,
            pltpu.SemaphoreType.DMA,
            pltpu.SemaphoreType.DMA,
        ],
        compiler_params=pltpu.CompilerParams(
            needs_layout_passes=False, use_tc_tiling_on_sc=True),
    )


@functools.cache
def _build(batch):
    assert batch % (_NW * _C) == 0
    b_per_w = batch // _NW
    nblk = b_per_w // _C
    mesh = plsc.VectorSubcoreMesh(core_axis_name="c", subcore_axis_name="s",
                                  num_cores=_NC, num_subcores=_NS)

    def body(x_hbm, lin_hbm, emb_hbm, bias_hbm, off_hbm, out_hbm,
             xv0, xv1, idxv0, idxv1, rowsv0, rowsv1, linv0, linv1,
             offv, biasv, outv, semx0, semx1, semg0, semg1):
        cid = lax.axis_index("c")
        sid = lax.axis_index("s")
        wid = sid * _NC + cid
        base_row = wid * b_per_w
        xvs, idxvs = (xv0, xv1), (idxv0, idxv1)
        rowsvs, linvs = (rowsv0, rowsv1), (linv0, linv1)
        semxs, semgs = (semx0, semx1), (semg0, semg1)

        pltpu.sync_copy(off_hbm, offv)
        pltpu.sync_copy(bias_hbm, biasv)
        biasvec = biasv[...]
        iota = lax.iota(jnp.int32, 16)

        def x_cp(s, blk):
            return pltpu.make_async_copy(
                x_hbm.at[pl.ds((base_row + blk * _C) * _F, _IPB)],
                xvs[s], semxs[s])

        def gather_cps(s):
            cps = []
            for j in range(_NCHUNK):
                cps.append(pltpu.make_async_copy(
                    emb_hbm.at[idxvs[s].at[pl.ds(j * 128, 128)]],
                    rowsvs[s].at[pl.ds(j * 128, 128)], semgs[s]))
                cps.append(pltpu.make_async_copy(
                    lin_hbm.at[idxvs[s].at[pl.ds(j * 128, 128)]],
                    linvs[s].at[pl.ds(j * 128, 128)], semgs[s]))
            return cps

        def fetch(s, blk):
            blk = jnp.int32(blk)
            x_cp(s, blk).wait()
            xv, idxv = xvs[s], idxvs[s]

            @plsc.parallel_loop(0, _IPB // 16, unroll=8)
            def _(t):
                sl = pl.ds(t * 16, 16)
                idxv[sl] = xv[sl] + offv[sl]

            for c in gather_cps(s):
                c.start()

            @pl.when(blk + 2 < nblk)
            def _():
                x_cp(s, blk + 2).start()

        def pool(s, blk):
            rowsv, linv = rowsvs[s], linvs[s]
            for c in gather_cps(s):
                c.wait()
            for g in range(_C // 16):
                rbase = iota * _F + g * (16 * _F)

                linsum = jnp.zeros((16,), jnp.float32)
                for f in range(_F):
                    linsum = linsum + plsc.load_gather(linv, [rbase + f])

                def d_body(d, part):
                    dvec = (iota + d) & 15
                    s_ = jnp.zeros((16,), jnp.float32)
                    sq = jnp.zeros((16,), jnp.float32)
                    for f in range(_F):
                        e = plsc.load_gather(rowsv, [rbase + f, dvec])
                        s_ = s_ + e
                        sq = sq + e * e
                    return part + (s_ * s_ - sq)

                acc = lax.fori_loop(0, _D, d_body,
                                    jnp.zeros((16,), jnp.float32))

                z = biasvec + linsum + 0.5 * acc
                outv[pl.ds(blk * _C + g * 16, 16)] = 1.0 / (1.0 + jnp.exp(-z))

        x_cp(0, 0).start()
        x_cp(1, 1).start()
        fetch(0, 0)

        def pipe(i, carry):
            b0 = 2 * i
            fetch(1, b0 + 1)
            pool(0, b0)
            @pl.when(b0 + 2 < nblk)
            def _():
                fetch(0, b0 + 2)
            pool(1, b0 + 1)
            return carry

        lax.fori_loop(0, nblk // 2, pipe, jnp.int32(0))
        pltpu.sync_copy(outv, out_hbm.at[pl.ds(base_row, b_per_w)])

    return pl.kernel(
        body,
        out_type=jax.ShapeDtypeStruct((batch,), jnp.float32),
        mesh=mesh,
        scratch_types=[
            pltpu.VMEM((_IPB,), jnp.int32),
            pltpu.VMEM((_IPB,), jnp.int32),
            pltpu.VMEM((_IPB,), jnp.int32),
            pltpu.VMEM((_IPB,), jnp.int32),
            pltpu.VMEM((_IPB, _D), jnp.float32),
            pltpu.VMEM((_IPB, _D), jnp.float32),
            pltpu.VMEM((_IPB,), jnp.float32),
            pltpu.VMEM((_IPB,), jnp.float32),
            pltpu.VMEM((_IPB,), jnp.int32),
            pltpu.VMEM((16,), jnp.float32),
            pltpu.VMEM((512,), jnp.float32),
            pltpu.SemaphoreType.DMA,
            pltpu.SemaphoreType.DMA,
            pltpu.SemaphoreType.DMA,
            pltpu.SemaphoreType.DMA,
        ],
        compiler_params=pltpu.CompilerParams(
            needs_layout_passes=False, use_tc_tiling_on_sc=False),
    )


def kernel(x, linear_w, embed_w, bias):
    batch, nf = x.shape
    assert nf == _F
    x_flat = x.reshape(-1).astype(jnp.int32)
    lin_flat = linear_w.reshape(-1).astype(jnp.float32)
    bias16 = jnp.broadcast_to(bias.reshape(()), (16,)).astype(jnp.float32)
    off_tile = jnp.asarray(np.tile(_OFFSETS, _C))
    table = _build_repack()(embed_w.T).reshape(_ROWS_OUT, 16)
    out = _build(batch)(x_flat, lin_flat, table, bias16, off_tile)
    return out.reshape(batch, 1)
